# baseline (speedup 1.0000x reference)
import functools

import numpy as np
import jax
import jax.numpy as jnp
from jax import lax
from jax.experimental import pallas as pl
from jax.experimental.pallas import tpu as pltpu
from jax.experimental.pallas import tpu_sc as plsc

D_MODEL = 768
NUM_TOKENS = 100000
BATCH = 4
SEQ = 8192
N = BATCH * SEQ

LANES = 16
NUM_CORES = 2
NUM_SUBCORES = 16
NW = NUM_CORES * NUM_SUBCORES
S_PER_W = SEQ // NW
K = 32
N_CHUNKS = S_PER_W // K
N_ITERS = N_CHUNKS * BATCH


def _pe_table():
    pos = np.arange(SEQ, dtype=np.float32)
    j = (2.0 * np.arange(D_MODEL // 2, dtype=np.float32)).astype(np.float32)
    denom = np.power(np.float32(10000.0), j / np.float32(D_MODEL)).astype(np.float32)
    ang = pos[:, None] / denom[None, :]
    pe = np.zeros((SEQ, D_MODEL), dtype=np.float32)
    pe[:, 0::2] = np.sin(ang)
    pe[:, 1::2] = np.cos(ang)
    return pe


def _pe_packed_i32():
    import ml_dtypes
    pe = _pe_table()
    bits = pe.astype(ml_dtypes.bfloat16).view(np.uint16).astype(np.uint32)
    g = bits.reshape(SEQ, D_MODEL // 32, 2, 16)
    packed = g[:, :, 0, :] | (g[:, :, 1, :] << 16)
    return packed.reshape(SEQ * D_MODEL // 2).astype(np.int32)


_PE = _pe_packed_i32()

_mesh = plsc.VectorSubcoreMesh(core_axis_name="c", subcore_axis_name="s")


@functools.partial(
    pl.kernel,
    mesh=_mesh,
    out_type=jax.ShapeDtypeStruct((N, D_MODEL), jnp.float32),
    scratch_types=[
        pltpu.VMEM((BATCH, S_PER_W), jnp.int32),
        pltpu.VMEM((K * D_MODEL // 2,), jnp.int32),
        pltpu.VMEM((K, D_MODEL), jnp.float32),
        pltpu.VMEM((K, D_MODEL), jnp.float32),
        pltpu.VMEM((K, D_MODEL), jnp.float32),
        pltpu.VMEM((K, D_MODEL), jnp.float32),
        pltpu.SemaphoreType.DMA,
        pltpu.SemaphoreType.DMA,
        pltpu.SemaphoreType.DMA,
        pltpu.SemaphoreType.DMA,
        pltpu.SemaphoreType.DMA,
        pltpu.SemaphoreType.DMA,
        pltpu.SemaphoreType.DMA,
        pltpu.SemaphoreType.DMA,
    ],
)
def _emb(table_hbm, xflat_hbm, pe_hbm, out_hbm,
         idx_all, pe_v, r0, r1, r2, r3, g0, g1, g2, g3, o0, o1, o2, o3):
    wid = lax.axis_index("s") * NUM_CORES + lax.axis_index("c")
    s_base = wid * S_PER_W

    rows_v = (r0, r1, r2, r3)
    gsem = (g0, g1, g2, g3)
    osem = (o0, o1, o2, o3)

    for b in range(BATCH):
        pltpu.sync_copy(
            xflat_hbm.at[b, pl.ds(pl.multiple_of(s_base, S_PER_W), S_PER_W)],
            idx_all.at[b],
        )

    def idx_ref(ci, b):
        return idx_all.at[b, pl.ds(ci * K, K)]

    def start_gather(ci, b):
        return pltpu.async_copy(table_hbm.at[idx_ref(ci, b)], rows_v[b], gsem[b])

    def wait_gather(ci, b):
        pltpu.make_async_copy(table_hbm.at[idx_ref(ci, b)], rows_v[b], gsem[b]).wait()

    def out_slice(ci, b):
        return out_hbm.at[pl.ds(pl.multiple_of(b * SEQ + s_base + ci * K, K), K)]

    def load_pe(ci):
        half = D_MODEL // 2
        start = pl.multiple_of((s_base + ci * K) * half, K * half)
        pltpu.sync_copy(pe_hbm.at[pl.ds(start, K * half)], pe_v)

    load_pe(0)
    for b in range(BATCH):
        start_gather(0, b)

    def super_body(ci, carry):
        for b in range(BATCH):
            wait_gather(ci, b)

            def row_body(r, c2, _rv=rows_v[b]):
                base = pl.multiple_of(r * (D_MODEL // 2), LANES)
                hi_mask = jnp.int32(-65536)
                for blk in range(D_MODEL // 32 // 4):
                    ws = [pe_v[pl.ds(base + (blk * 4 + q) * LANES, LANES)]
                          for q in range(4)]
                    for q in range(4):
                        g = blk * 4 + q
                        pa = lax.bitcast_convert_type(ws[q] << 16, jnp.float32)
                        pb = lax.bitcast_convert_type(ws[q] & hi_mask, jnp.float32)
                        plsc.addupdate(_rv.at[r, pl.ds(g * 32, LANES)], pa)
                        plsc.addupdate(_rv.at[r, pl.ds(g * 32 + LANES, LANES)], pb)
                return c2

            lax.fori_loop(0, K, row_body, 0)
            pltpu.async_copy(rows_v[b], out_slice(ci, b), osem[b])
            if b >= 1:
                @pl.when(ci < N_CHUNKS - 1)
                def _refill(_b=b):
                    pltpu.make_async_copy(
                        rows_v[_b - 1], out_slice(ci, _b - 1), osem[_b - 1]).wait()
                    start_gather(ci + 1, _b - 1)

        @pl.when(ci < N_CHUNKS - 1)
        def _tail():
            load_pe(ci + 1)
            pltpu.make_async_copy(
                rows_v[BATCH - 1], out_slice(ci, BATCH - 1), osem[BATCH - 1]).wait()
            start_gather(ci + 1, BATCH - 1)
        return carry

    lax.fori_loop(0, N_CHUNKS, super_body, 0)
    for b in range(BATCH):
        pltpu.make_async_copy(rows_v[b], out_slice(N_CHUNKS - 1, b), osem[b]).wait()


def kernel(x, token_embeddings):
    pe = jnp.asarray(_PE)
    out = _emb(token_embeddings, x.astype(jnp.int32), pe)
    return out.reshape(BATCH, SEQ, D_MODEL)


# speedup vs baseline: 1.0150x; 1.0150x over previous
import functools

import numpy as np
import jax
import jax.numpy as jnp
from jax import lax
from jax.experimental import pallas as pl
from jax.experimental.pallas import tpu as pltpu
from jax.experimental.pallas import tpu_sc as plsc

D_MODEL = 768
NUM_TOKENS = 100000
BATCH = 4
SEQ = 8192
N = BATCH * SEQ

LANES = 16
NUM_CORES = 2
NUM_SUBCORES = 16
NW = NUM_CORES * NUM_SUBCORES
S_PER_W = SEQ // NW
K = 32
N_CHUNKS = S_PER_W // K
N_ITERS = N_CHUNKS * BATCH


def _pe_table():
    pos = np.arange(SEQ, dtype=np.float32)
    j = (2.0 * np.arange(D_MODEL // 2, dtype=np.float32)).astype(np.float32)
    denom = np.power(np.float32(10000.0), j / np.float32(D_MODEL)).astype(np.float32)
    ang = pos[:, None] / denom[None, :]
    pe = np.zeros((SEQ, D_MODEL), dtype=np.float32)
    pe[:, 0::2] = np.sin(ang)
    pe[:, 1::2] = np.cos(ang)
    return pe


def _pe_packed_i32():
    import ml_dtypes
    pe = _pe_table()
    bits = pe.astype(ml_dtypes.bfloat16).view(np.uint16).astype(np.uint32)
    g = bits.reshape(SEQ, D_MODEL // 32, 2, 16)
    packed = g[:, :, 0, :] | (g[:, :, 1, :] << 16)
    return packed.reshape(SEQ * D_MODEL // 2).astype(np.int32)


_PE = _pe_packed_i32()

_mesh = plsc.VectorSubcoreMesh(core_axis_name="c", subcore_axis_name="s")


@functools.partial(
    pl.kernel,
    mesh=_mesh,
    out_type=jax.ShapeDtypeStruct((N, D_MODEL), jnp.float32),
    scratch_types=[
        pltpu.VMEM((BATCH, S_PER_W), jnp.int32),
        pltpu.VMEM((K * D_MODEL // 2,), jnp.int32),
        pltpu.VMEM((K, D_MODEL), jnp.float32),
        pltpu.VMEM((K, D_MODEL), jnp.float32),
        pltpu.VMEM((K, D_MODEL), jnp.float32),
        pltpu.VMEM((K, D_MODEL), jnp.float32),
        pltpu.SemaphoreType.DMA,
        pltpu.SemaphoreType.DMA,
        pltpu.SemaphoreType.DMA,
        pltpu.SemaphoreType.DMA,
        pltpu.SemaphoreType.DMA,
        pltpu.SemaphoreType.DMA,
        pltpu.SemaphoreType.DMA,
        pltpu.SemaphoreType.DMA,
    ],
)
def _emb(table_hbm, xflat_hbm, pe_hbm, out_hbm,
         idx_all, pe_v, r0, r1, r2, r3, g0, g1, g2, g3, o0, o1, o2, o3):
    wid = lax.axis_index("s") * NUM_CORES + lax.axis_index("c")
    s_base = wid * S_PER_W

    rows_v = (r0, r1, r2, r3)
    gsem = (g0, g1, g2, g3)
    osem = (o0, o1, o2, o3)

    pltpu.sync_copy(
        xflat_hbm.at[:, pl.ds(pl.multiple_of(s_base, S_PER_W), S_PER_W)],
        idx_all,
    )

    def idx_ref(ci, b):
        return idx_all.at[b, pl.ds(ci * K, K)]

    def start_gather(ci, b):
        return pltpu.async_copy(table_hbm.at[idx_ref(ci, b)], rows_v[b], gsem[b])

    def wait_gather(ci, b):
        pltpu.make_async_copy(table_hbm.at[idx_ref(ci, b)], rows_v[b], gsem[b]).wait()

    def out_slice(ci, b):
        return out_hbm.at[pl.ds(pl.multiple_of(b * SEQ + s_base + ci * K, K), K)]

    def load_pe(ci):
        half = D_MODEL // 2
        start = pl.multiple_of((s_base + ci * K) * half, K * half)
        pltpu.sync_copy(pe_hbm.at[pl.ds(start, K * half)], pe_v)

    load_pe(0)
    for b in range(BATCH):
        start_gather(0, b)

    def super_body(ci, carry):
        for b in range(BATCH):
            wait_gather(ci, b)

            def row_body(r, c2, _rv=rows_v[b]):
                base = pl.multiple_of(r * (D_MODEL // 2), LANES)
                hi_mask = jnp.int32(-65536)
                for blk in range(D_MODEL // 32 // 4):
                    ws = [pe_v[pl.ds(base + (blk * 4 + q) * LANES, LANES)]
                          for q in range(4)]
                    for q in range(4):
                        g = blk * 4 + q
                        pa = lax.bitcast_convert_type(ws[q] << 16, jnp.float32)
                        pb = lax.bitcast_convert_type(ws[q] & hi_mask, jnp.float32)
                        plsc.addupdate(_rv.at[r, pl.ds(g * 32, LANES)], pa)
                        plsc.addupdate(_rv.at[r, pl.ds(g * 32 + LANES, LANES)], pb)
                return c2

            lax.fori_loop(0, K, row_body, 0)
            pltpu.async_copy(rows_v[b], out_slice(ci, b), osem[b])
            if b >= 1:
                @pl.when(ci < N_CHUNKS - 1)
                def _refill(_b=b):
                    pltpu.make_async_copy(
                        rows_v[_b - 1], out_slice(ci, _b - 1), osem[_b - 1]).wait()
                    start_gather(ci + 1, _b - 1)

        @pl.when(ci < N_CHUNKS - 1)
        def _tail():
            load_pe(ci + 1)
            pltpu.make_async_copy(
                rows_v[BATCH - 1], out_slice(ci, BATCH - 1), osem[BATCH - 1]).wait()
            start_gather(ci + 1, BATCH - 1)
        return carry

    lax.fori_loop(0, N_CHUNKS, super_body, 0)
    for b in range(BATCH):
        pltpu.make_async_copy(rows_v[b], out_slice(N_CHUNKS - 1, b), osem[b]).wait()


def kernel(x, token_embeddings):
    pe = jnp.asarray(_PE)
    out = _emb(token_embeddings, x.astype(jnp.int32), pe)
    return out.reshape(BATCH, SEQ, D_MODEL)
